# baseline (speedup 1.0000x reference)
import functools

import jax
import jax.numpy as jnp
from jax import lax
from jax.experimental import pallas as pl
from jax.experimental.pallas import tpu as pltpu
from jax.experimental.pallas import tpu_sc as plsc

_K = 8
_L = 16
_CH = 16
_GRP = 16
_AUNR = 2
_NC = 2
_NS = 16
_NW = _NC * _NS


def _sort_desc(v):
    s, _ = plsc.sort_key_val(v, v, descending=True)
    return s


def _merge_desc(a, b):
    t = jnp.maximum(a, lax.rev(b, (0,)))
    return _sort_desc(t)


def _insert(ms, v):
    out = []
    t = v
    for mk in ms:
        out.append(jnp.maximum(mk, t))
        t = jnp.minimum(mk, t)
    return tuple(out)


def _tree_max(vs):
    while len(vs) > 1:
        vs = [jnp.maximum(vs[2 * j], vs[2 * j + 1]) for j in range(len(vs) // 2)]
    return vs[0]


def kernel(scores):
    rows, n = scores.shape
    rows_per_w = rows // _NW
    nvec = n // _L
    nch = nvec // _CH
    ngrp = nch // _GRP
    mesh = plsc.VectorSubcoreMesh(core_axis_name="c", subcore_axis_name="s")

    @functools.partial(
        pl.kernel,
        out_type=jax.ShapeDtypeStruct((rows * _K,), jnp.float32),
        mesh=mesh,
        scratch_types=[
            pltpu.VMEM((n,), jnp.float32),
            pltpu.VMEM((n,), jnp.float32),
            pltpu.VMEM((_L * (nch + 1),), jnp.float32),
            pltpu.VMEM((nch + _L,), jnp.int32),
            pltpu.VMEM((2 * _L,), jnp.float32),
            pltpu.VMEM((rows_per_w * _K,), jnp.float32),
            pltpu.SemaphoreType.DMA,
            pltpu.SemaphoreType.DMA,
            pltpu.SemaphoreType.DMA,
            pltpu.SemaphoreType.DMA,
            pltpu.SemaphoreType.DMA,
        ],
        compiler_params=pltpu.CompilerParams(needs_layout_passes=False),
    )
    def _topk(
        scores_hbm,
        out_hbm,
        rowbuf0,
        rowbuf1,
        cmaxt,
        hitbuf,
        packbuf,
        outbuf,
        sem0,
        sem1,
        semq1,
        semq2,
        semq3,
    ):
        cid = lax.axis_index("c")
        sid = lax.axis_index("s")
        base = (sid * _NC + cid) * rows_per_w
        bufs = (rowbuf0, rowbuf1)
        sems = (sem0, sem1)

        lane = lax.iota(jnp.int32, _L)
        scat_base = lane * (nch + 1)
        neg = jnp.full((_L,), -jnp.inf, jnp.float32)
        init = tuple(neg for _ in range(_K))

        nq = 4
        qlen = n // nq
        qsems = (sem0, semq1, semq2, semq3)
        row0 = scores_hbm.at[base]
        for i in range(nq):
            pltpu.make_async_copy(
                row0.at[pl.ds(i * qlen, qlen)],
                rowbuf0.at[pl.ds(i * qlen, qlen)],
                qsems[i],
            ).start()
        pltpu.make_async_copy(scores_hbm.at[base + 1], rowbuf1, sem1).start()

        def make_chunk_max(rb):
            def chunk_max(cb, m):
                cms = []
                for h in range(_AUNR):
                    c = cb * _AUNR + h
                    bv = c * (_CH * _L)
                    cm = _tree_max(
                        [rb[pl.ds(bv + j * _L, _L)] for j in range(_CH)]
                    )
                    plsc.store_scatter(cmaxt, [scat_base + c], cm)
                    cms.append(cm)
                return jnp.maximum(m, _tree_max(cms))

            return chunk_max

        def finish_row(rb, m):
            sm = _sort_desc(m)
            tau = lax.reduce_max(jnp.where(lane >= _K - 1, sm, -jnp.inf), (0,))

            def find_hits(g, ptr):
                acc = _tree_max(
                    [
                        cmaxt[pl.ds(l * (nch + 1) + g * _GRP, _L)]
                        for l in range(_L)
                    ]
                )
                hits = acc >= tau
                pop = plsc.all_reduce_population_count(hits)
                plsc.store_compressed(
                    hitbuf.at[pl.ds(ptr, _L)], g * _GRP + lane, mask=hits
                )
                return ptr + pop[0]

            with jax.named_scope("passB1"):
                nhit = lax.fori_loop(0, ngrp, find_hits, jnp.int32(0))

            def insert_hit(i, ms):
                c = hitbuf[pl.ds(i, _L)][0]
                bv = c * (_CH * _L)
                out = ms
                for j in range(_CH):
                    out = _insert(out, rb[pl.ds(bv + j * _L, _L)])
                return out

            with jax.named_scope("passB2"):
                ms = lax.fori_loop(0, nhit, insert_hit, init)

            with jax.named_scope("select"):
                s = [_sort_desc(mv) for mv in ms]
                s = [_merge_desc(s[2 * j], s[2 * j + 1]) for j in range(4)]
                s = [_merge_desc(s[0], s[1]), _merge_desc(s[2], s[3])]
                return _merge_desc(s[0], s[1])

        finals = []
        qch = nch // nq
        for r in range(rows_per_w):
            rb = bufs[r % 2]
            chunk_max = make_chunk_max(rb)
            if r == 0:
                m = neg
                for i in range(nq):
                    with jax.named_scope("dma_wait"):
                        pltpu.make_async_copy(
                            row0.at[pl.ds(i * qlen, qlen)],
                            rowbuf0.at[pl.ds(i * qlen, qlen)],
                            qsems[i],
                        ).wait()
                    with jax.named_scope("passA"):
                        m = lax.fori_loop(
                            i * qch // _AUNR,
                            (i + 1) * qch // _AUNR,
                            chunk_max,
                            m,
                        )
            else:
                with jax.named_scope("dma_wait"):
                    pltpu.make_async_copy(
                        scores_hbm.at[base + r], rb, sems[r % 2]
                    ).wait()
                with jax.named_scope("passA"):
                    m = lax.fori_loop(0, nch // _AUNR, chunk_max, neg)
            finals.append(finish_row(rb, m))
            if r + 2 < rows_per_w:
                pltpu.make_async_copy(
                    scores_hbm.at[base + r + 2], rb, sems[r % 2]
                ).start()

        for t in range(rows_per_w // 2):
            packbuf[pl.ds(0, _L)] = finals[2 * t]
            packbuf[pl.ds(_K, _L)] = finals[2 * t + 1]
            outbuf[pl.ds(t * _L, _L)] = packbuf[pl.ds(0, _L)]

        pltpu.sync_copy(outbuf, out_hbm.at[pl.ds(base * _K, rows_per_w * _K)])

    out = _topk(scores)
    return out.reshape(rows, _K)


# speedup vs baseline: 1.0748x; 1.0748x over previous
import functools

import jax
import jax.numpy as jnp
from jax import lax
from jax.experimental import pallas as pl
from jax.experimental.pallas import tpu as pltpu
from jax.experimental.pallas import tpu_sc as plsc

_K = 8
_L = 16
_CH = 16
_GRP = 16
_AUNR = 2
_NC = 2
_NS = 16
_NW = _NC * _NS


def _sort_desc(v):
    s, _ = plsc.sort_key_val(v, v, descending=True)
    return s


def _merge_desc(a, b):
    t = jnp.maximum(a, lax.rev(b, (0,)))
    return _sort_desc(t)


def _insert(ms, v):
    out = []
    t = v
    for mk in ms:
        out.append(jnp.maximum(mk, t))
        t = jnp.minimum(mk, t)
    return tuple(out)


def _tree_max(vs):
    while len(vs) > 1:
        vs = [jnp.maximum(vs[2 * j], vs[2 * j + 1]) for j in range(len(vs) // 2)]
    return vs[0]


def kernel(scores):
    rows, n = scores.shape
    rows_per_w = rows // _NW
    nvec = n // _L
    nch = nvec // _CH
    ngrp = nch // _GRP
    mesh = plsc.VectorSubcoreMesh(core_axis_name="c", subcore_axis_name="s")

    @functools.partial(
        pl.kernel,
        out_type=jax.ShapeDtypeStruct((rows * _K,), jnp.float32),
        mesh=mesh,
        scratch_types=[
            pltpu.VMEM((n,), jnp.float32),
            pltpu.VMEM((n,), jnp.float32),
            pltpu.VMEM((_L * (nch + 1),), jnp.float32),
            pltpu.VMEM((nch + _L,), jnp.int32),
            pltpu.VMEM((2 * _L,), jnp.float32),
            pltpu.VMEM((rows_per_w * _K,), jnp.float32),
            pltpu.SemaphoreType.DMA,
            pltpu.SemaphoreType.DMA,
        ],
        compiler_params=pltpu.CompilerParams(needs_layout_passes=False),
    )
    def _topk(
        scores_hbm,
        out_hbm,
        rowbuf0,
        rowbuf1,
        cmaxt,
        hitbuf,
        packbuf,
        outbuf,
        sem0,
        sem1,
    ):
        cid = lax.axis_index("c")
        sid = lax.axis_index("s")
        base = (sid * _NC + cid) * rows_per_w
        bufs = (rowbuf0, rowbuf1)
        sems = (sem0, sem1)

        lane = lax.iota(jnp.int32, _L)
        scat_base = lane * (nch + 1)
        neg = jnp.full((_L,), -jnp.inf, jnp.float32)

        npair = rows_per_w // 2
        pltpu.make_async_copy(scores_hbm.at[base], rowbuf0, sem0).start()

        def row_pair(t, carry):
            finals = []
            for half in range(2):
                r = t * 2 + half
                rb = bufs[half]
                with jax.named_scope("dma_wait"):
                    pltpu.make_async_copy(
                        scores_hbm.at[base + r], rb, sems[half]
                    ).wait()
                if half == 0:

                    @pl.when(t == 0)
                    def _():
                        pltpu.make_async_copy(
                            scores_hbm.at[base + 1], rowbuf1, sem1
                        ).start()

                init = tuple(neg for _ in range(_K))

                def chunk_max(cb, m):
                    cms = []
                    for h in range(_AUNR):
                        c = cb * _AUNR + h
                        bv = c * (_CH * _L)
                        cm = _tree_max(
                            [rb[pl.ds(bv + j * _L, _L)] for j in range(_CH)]
                        )
                        plsc.store_scatter(cmaxt, [scat_base + c], cm)
                        cms.append(cm)
                    return jnp.maximum(m, _tree_max(cms))

                with jax.named_scope("passA"):
                    m = lax.fori_loop(0, nch // _AUNR, chunk_max, neg)

                sm = _sort_desc(m)
                tau = lax.reduce_max(
                    jnp.where(lane >= _K - 1, sm, -jnp.inf), (0,)
                )

                def find_hits(g, ptr):
                    acc = _tree_max(
                        [
                            cmaxt[pl.ds(l * (nch + 1) + g * _GRP, _L)]
                            for l in range(_L)
                        ]
                    )
                    hits = acc >= tau
                    pop = plsc.all_reduce_population_count(hits)
                    plsc.store_compressed(
                        hitbuf.at[pl.ds(ptr, _L)], g * _GRP + lane, mask=hits
                    )
                    return ptr + pop[0]

                with jax.named_scope("passB1"):
                    nhit = lax.fori_loop(0, ngrp, find_hits, jnp.int32(0))

                def insert_hit(i, ms):
                    c = hitbuf[pl.ds(i, _L)][0]
                    bv = c * (_CH * _L)
                    out = ms
                    for j in range(_CH):
                        out = _insert(out, rb[pl.ds(bv + j * _L, _L)])
                    return out

                with jax.named_scope("passB2"):
                    ms = lax.fori_loop(0, nhit, insert_hit, init)

                @pl.when(t < npair - 1)
                def _():
                    pltpu.make_async_copy(
                        scores_hbm.at[base + r + 2], rb, sems[half]
                    ).start()

                with jax.named_scope("select"):
                    s = [_sort_desc(mv) for mv in ms]
                    s = [_merge_desc(s[2 * j], s[2 * j + 1]) for j in range(4)]
                    s = [_merge_desc(s[0], s[1]), _merge_desc(s[2], s[3])]
                    finals.append(_merge_desc(s[0], s[1]))

            packbuf[pl.ds(0, _L)] = finals[0]
            packbuf[pl.ds(_K, _L)] = finals[1]
            outbuf[pl.ds(t * _L, _L)] = packbuf[pl.ds(0, _L)]
            return carry

        lax.fori_loop(0, npair, row_pair, jnp.int32(0))
        pltpu.sync_copy(outbuf, out_hbm.at[pl.ds(base * _K, rows_per_w * _K)])

    out = _topk(scores)
    return out.reshape(rows, _K)
